# Initial kernel scaffold; baseline (speedup 1.0000x reference)
import functools
import jax
import jax.numpy as jnp
from jax import lax
from jax.experimental import pallas as pl
from jax.experimental.pallas import tpu as pltpu, tpu_sc as plsc

N = 10000
E = 160000
H = 128
H2 = 2 * H

NC = 2
NS = 16
NW = NC * NS
CHUNK = 125
NCHUNK = E // CHUNK
CPW = NCHUNK // NW
ROWS_PER_SUB = N // NS

BN = 1000
BE = 2000

_mesh = plsc.VectorSubcoreMesh(core_axis_name="c", subcore_axis_name="s")


def _sc_gather_body(a_hbm, b_hbm, dsti, srci, out_hbm,
                    idx_d, idx_s, buf_a, buf_b, sem_a, sem_b):
    c = lax.axis_index("c")
    s = lax.axis_index("s")
    wid = s * NC + c

    def chunk_body(j, carry):
        ch = wid * CPW + j
        pltpu.sync_copy(dsti.at[ch], idx_d)
        pltpu.sync_copy(srci.at[ch], idx_s)
        cp_a = pltpu.async_copy(a_hbm.at[idx_d], buf_a, sem_a)
        cp_b = pltpu.async_copy(b_hbm.at[idx_s], buf_b, sem_b)
        cp_a.wait()
        cp_b.wait()

        def row_body(i, carry2):
            for g in range(H2 // 16):
                sl = pl.ds(g * 16, 16)
                buf_a[i, sl] = buf_a[i, sl] + buf_b[i, sl]
            return carry2

        lax.fori_loop(0, CHUNK, row_body, 0)
        pltpu.sync_copy(buf_a, out_hbm.at[pl.ds(ch * CHUNK, CHUNK)])
        return carry

    lax.fori_loop(0, CPW, chunk_body, 0)


_sc_gather = pl.kernel(
    _sc_gather_body,
    out_type=jax.ShapeDtypeStruct((E, H2), jnp.float32),
    mesh=_mesh,
    scratch_types=[
        pltpu.VMEM((CHUNK,), jnp.int32),
        pltpu.VMEM((CHUNK,), jnp.int32),
        pltpu.VMEM((CHUNK, H2), jnp.float32),
        pltpu.VMEM((CHUNK, H2), jnp.float32),
        pltpu.SemaphoreType.DMA,
        pltpu.SemaphoreType.DMA,
    ],
)


def _make_sc_scatter(with_counts):
    def body(m2_hbm, dsti, z128, z16, agg_out, cnt_out,
             idx, mbuf, ones_v, agg_sh, cnt_sh):
        c = lax.axis_index("c")
        s = lax.axis_index("s")
        wid = s * NC + c

        @pl.when(s == 0)
        def _zero():
            pltpu.sync_copy(z128, agg_sh)
            if with_counts:
                pltpu.sync_copy(z16, cnt_sh)

        if with_counts:
            def ones_body(i, carry):
                ones_v[i, :] = jnp.full((16,), 1.0, jnp.float32)
                return carry
            lax.fori_loop(0, CHUNK, ones_body, 0)

        plsc.subcore_barrier()

        def chunk_body(j, carry):
            ch = wid * CPW + j
            pltpu.sync_copy(dsti.at[ch], idx)
            pltpu.sync_copy(m2_hbm.at[pl.ds(ch * CHUNK, CHUNK)], mbuf)
            pltpu.sync_copy(mbuf, agg_sh.at[idx], add=True)
            if with_counts:
                pltpu.sync_copy(ones_v, cnt_sh.at[idx], add=True)
            return carry

        lax.fori_loop(0, CPW, chunk_body, 0)
        plsc.subcore_barrier()

        rows = pl.ds(s * ROWS_PER_SUB, ROWS_PER_SUB)
        pltpu.sync_copy(agg_sh.at[rows], agg_out.at[c, rows])
        if with_counts:
            pltpu.sync_copy(cnt_sh.at[rows], cnt_out.at[c, rows])

    return pl.kernel(
        body,
        out_type=[
            jax.ShapeDtypeStruct((NC, N, H), jnp.float32),
            jax.ShapeDtypeStruct((NC, N, 16), jnp.float32),
        ],
        mesh=_mesh,
        scratch_types=[
            pltpu.VMEM((CHUNK,), jnp.int32),
            pltpu.VMEM((CHUNK, H), jnp.float32),
            pltpu.VMEM((CHUNK, 16), jnp.float32),
            pltpu.VMEM_SHARED((N, H), jnp.float32),
            pltpu.VMEM_SHARED((N, 16), jnp.float32),
        ],
    )


_sc_scatter_cnt = _make_sc_scatter(True)
_sc_scatter = _make_sc_scatter(False)


def _ln_relu(z, g, b):
    mu = jnp.mean(z, axis=-1, keepdims=True)
    zc = z - mu
    var = jnp.mean(zc * zc, axis=-1, keepdims=True)
    return jax.nn.relu(zc * jax.lax.rsqrt(var + 1e-5) * g + b)


def _dot(a, b):
    return jnp.dot(a, b, preferred_element_type=jnp.float32)


def _node_pre_body(feats_ref, nw_ref, nb_ref, wa_ref, wb_ref,
                   h_ref, a_ref, b_ref):
    h = _dot(feats_ref[...], nw_ref[...]) + nb_ref[...]
    h_ref[...] = h
    a_ref[...] = _dot(h, wa_ref[...])
    b_ref[...] = _dot(h, wb_ref[...])


def _node_pre(feats, nw, nb, wa, wb):
    full = lambda shape: pl.BlockSpec(shape, lambda i: (0,) * len(shape))
    return pl.pallas_call(
        _node_pre_body,
        grid=(N // BN,),
        in_specs=[
            pl.BlockSpec((BN, H), lambda i: (i, 0)),
            full((H, H)), full((1, H)), full((H, H2)), full((H, H2)),
        ],
        out_specs=[
            pl.BlockSpec((BN, H), lambda i: (i, 0)),
            pl.BlockSpec((BN, H2), lambda i: (i, 0)),
            pl.BlockSpec((BN, H2), lambda i: (i, 0)),
        ],
        out_shape=[
            jax.ShapeDtypeStruct((N, H), jnp.float32),
            jax.ShapeDtypeStruct((N, H2), jnp.float32),
            jax.ShapeDtypeStruct((N, H2), jnp.float32),
        ],
    )(feats, nw, nb, wa, wb)


def _edge_mlp_body(g_ref, ea_ref, c_ref, cb_ref, w2_ref, b2_ref,
                   g1_ref, be1_ref, g2_ref, be2_ref, out_ref):
    z = g_ref[...] + _dot(ea_ref[...], c_ref[...]) + cb_ref[...]
    m = _ln_relu(z, g1_ref[...], be1_ref[...])
    m2 = _dot(m, w2_ref[...]) + b2_ref[...]
    out_ref[...] = _ln_relu(m2, g2_ref[...], be2_ref[...])


def _edge_mlp(g, ea, cmat, cbias, w2, b2, g1, be1, g2, be2):
    full = lambda shape: pl.BlockSpec(shape, lambda i: (0,) * len(shape))
    return pl.pallas_call(
        _edge_mlp_body,
        grid=(E // BE,),
        in_specs=[
            pl.BlockSpec((BE, H2), lambda i: (i, 0)),
            pl.BlockSpec((BE, 16), lambda i: (i, 0)),
            full((16, H2)), full((1, H2)), full((H2, H)), full((1, H)),
            full((1, H2)), full((1, H2)), full((1, H)), full((1, H)),
        ],
        out_specs=pl.BlockSpec((BE, H), lambda i: (i, 0)),
        out_shape=jax.ShapeDtypeStruct((E, H), jnp.float32),
    )(g, ea, cmat, cbias, w2, b2, g1, be1, g2, be2)


def _make_post_body(with_ab):
    def body(*refs):
        if with_ab:
            (aggp_ref, cntp_ref, h_ref, pw_ref, pb_ref, ng_ref, nb_ref,
             wa_ref, wb_ref, out_ref, a_ref, b_ref) = refs
        else:
            (aggp_ref, cntp_ref, h_ref, pw_ref, pb_ref, ng_ref, nb_ref,
             out_ref) = refs
        agg = aggp_ref[0] + aggp_ref[1]
        cnt = cntp_ref[0, :, 0] + cntp_ref[1, :, 0]
        agg = agg / jnp.maximum(cnt, 1.0)[:, None]
        o = _dot(agg, pw_ref[...]) + pb_ref[...]
        hn = _ln_relu(o, ng_ref[...], nb_ref[...]) + h_ref[...]
        out_ref[...] = hn
        if with_ab:
            a_ref[...] = _dot(hn, wa_ref[...])
            b_ref[...] = _dot(hn, wb_ref[...])
    return body


def _post(aggp, cntp, h, pw, pb, ng, nb, wa=None, wb=None):
    with_ab = wa is not None
    full = lambda shape: pl.BlockSpec(shape, lambda i: (0,) * len(shape))
    in_specs = [
        pl.BlockSpec((NC, BN, H), lambda i: (0, i, 0)),
        pl.BlockSpec((NC, BN, 16), lambda i: (0, i, 0)),
        pl.BlockSpec((BN, H), lambda i: (i, 0)),
        full((H, H)), full((1, H)), full((1, H)), full((1, H)),
    ]
    out_specs = [pl.BlockSpec((BN, H), lambda i: (i, 0))]
    out_shape = [jax.ShapeDtypeStruct((N, H), jnp.float32)]
    args = [aggp, cntp, h, pw, pb, ng, nb]
    if with_ab:
        in_specs += [full((H, H2)), full((H, H2))]
        out_specs += [pl.BlockSpec((BN, H2), lambda i: (i, 0)),
                      pl.BlockSpec((BN, H2), lambda i: (i, 0))]
        out_shape += [jax.ShapeDtypeStruct((N, H2), jnp.float32),
                      jax.ShapeDtypeStruct((N, H2), jnp.float32)]
        args += [wa, wb]
    out = pl.pallas_call(
        _make_post_body(with_ab),
        grid=(N // BN,),
        in_specs=in_specs,
        out_specs=out_specs,
        out_shape=out_shape,
    )(*args)
    return out


def kernel(x, pos, edge_attr, params, edge_index, batch):
    feats = jnp.concatenate([x, pos], axis=1)
    src = edge_index[0]
    dst = edge_index[1]
    dsti = dst.reshape(NCHUNK, CHUNK)
    srci = src.reshape(NCHUNK, CHUNK)

    row = lambda v: v.reshape(1, -1)
    wa, wb, cmat, cbias = [], [], [], []
    for lp in params['layers']:
        w1 = lp['W1']
        w1a, w1b, w1c = w1[:H], w1[H:2 * H], w1[2 * H:]
        wa.append(w1a - w1b)
        wb.append(w1b)
        cmat.append(params['edge_W'] @ w1c)
        cbias.append(row(params['edge_b'] @ w1c + lp['b1']))

    h, a, b = _node_pre(feats, params['node_W'], row(params['node_b']),
                        wa[0], wb[0])

    z128 = jnp.zeros((N, H), jnp.float32)
    z16 = jnp.zeros((N, 16), jnp.float32)

    cntp = None
    for li, lp in enumerate(params['layers']):
        g = _sc_gather(a, b, dsti, srci)
        m2 = _edge_mlp(g, edge_attr, cmat[li], cbias[li], lp['W2'],
                       row(lp['b2']), row(lp['g1']), row(lp['be1']),
                       row(lp['g2']), row(lp['be2']))
        if li == 0:
            aggp, cntp = _sc_scatter_cnt(m2, dsti, z128, z16)
            h, a, b = _post(aggp, cntp, h, lp['pW'], row(lp['pb']),
                            row(lp['ng']), row(lp['nb']),
                            wa[1], wb[1])
        else:
            aggp, _ = _sc_scatter(m2, dsti, z128, z16)
            h = _post(aggp, cntp, h, lp['pW'], row(lp['pb']),
                      row(lp['ng']), row(lp['nb']))[0]
    return h


# speedup vs baseline: 2.8745x; 2.8745x over previous
import functools
import jax
import jax.numpy as jnp
from jax import lax
from jax.experimental import pallas as pl
from jax.experimental.pallas import tpu as pltpu, tpu_sc as plsc

N = 10000
NP = 10240
E = 160000
H = 128
H2 = 2 * H

NC = 2
NS = 16
NW = NC * NS
CHUNK = 128
NCHUNK = E // CHUNK
CPW = -(-NCHUNK // NW)
ROWS_PER_SUB = NP // NS

BN = 1000
BE = 2000

_mesh = plsc.VectorSubcoreMesh(core_axis_name="c", subcore_axis_name="s")


def _sc_gather_body(a_hbm, b_hbm, dsti, srci, out_hbm,
                    idx_d, idx_s, buf_a, buf_b, sem_a, sem_b):
    c = lax.axis_index("c")
    s = lax.axis_index("s")
    wid = s * NC + c

    def chunk_body(j, carry):
        ch = j * NW + wid

        @pl.when(ch < NCHUNK)
        def _():
            pltpu.sync_copy(dsti.at[ch, 0], idx_d)
            pltpu.sync_copy(srci.at[ch, 0], idx_s)
            cp_a = pltpu.async_copy(a_hbm.at[idx_d], buf_a, sem_a)
            cp_b = pltpu.async_copy(b_hbm.at[idx_s], buf_b, sem_b)
            cp_a.wait()
            cp_b.wait()

            def row_body(i, carry2):
                for g in range(H2 // 16):
                    sl = pl.ds(g * 16, 16)
                    buf_a[i, sl] = buf_a[i, sl] + buf_b[i, sl]
                return carry2

            lax.fori_loop(0, CHUNK, row_body, 0)
            pltpu.sync_copy(buf_a, out_hbm.at[pl.ds(ch * CHUNK, CHUNK)])

        return carry

    lax.fori_loop(0, CPW, chunk_body, 0)


_sc_gather = pl.kernel(
    _sc_gather_body,
    out_type=jax.ShapeDtypeStruct((E, H2), jnp.float32),
    mesh=_mesh,
    scratch_types=[
        pltpu.VMEM((CHUNK,), jnp.int32),
        pltpu.VMEM((CHUNK,), jnp.int32),
        pltpu.VMEM((CHUNK, H2), jnp.float32),
        pltpu.VMEM((CHUNK, H2), jnp.float32),
        pltpu.SemaphoreType.DMA,
        pltpu.SemaphoreType.DMA,
    ],
)


def _sc_scatter_body(m2_hbm, dsti, z128, agg_out, idx, mbuf, agg_sh):
    c = lax.axis_index("c")
    s = lax.axis_index("s")
    wid = s * NC + c

    @pl.when(s == 0)
    def _zero():
        pltpu.sync_copy(z128, agg_sh)

    plsc.subcore_barrier()

    def chunk_body(j, carry):
        ch = j * NW + wid

        @pl.when(ch < NCHUNK)
        def _():
            pltpu.sync_copy(dsti.at[ch, 0], idx)
            pltpu.sync_copy(m2_hbm.at[pl.ds(ch * CHUNK, CHUNK)], mbuf)
            pltpu.sync_copy(mbuf, agg_sh.at[idx], add=True)

        return carry

    lax.fori_loop(0, CPW, chunk_body, 0)
    plsc.subcore_barrier()

    rows = pl.ds(s * ROWS_PER_SUB, ROWS_PER_SUB)
    pltpu.sync_copy(agg_sh.at[rows], agg_out.at[c, rows])


_sc_scatter = pl.kernel(
    _sc_scatter_body,
    out_type=jax.ShapeDtypeStruct((NC, NP, H), jnp.float32),
    mesh=_mesh,
    scratch_types=[
        pltpu.VMEM((CHUNK,), jnp.int32),
        pltpu.VMEM((CHUNK, H), jnp.float32),
        pltpu.VMEM_SHARED((NP, H), jnp.float32),
    ],
)


def _sc_count_body(dsti, z128, cnt_out, idx, ones_v, cnt_sh):
    c = lax.axis_index("c")
    s = lax.axis_index("s")
    wid = s * NC + c

    @pl.when(s == 0)
    def _zero():
        pltpu.sync_copy(z128, cnt_sh)

    def ones_body(i, carry):
        for g in range(H // 16):
            ones_v[i, pl.ds(g * 16, 16)] = jnp.full((16,), 1.0, jnp.float32)
        return carry
    lax.fori_loop(0, CHUNK, ones_body, 0)

    plsc.subcore_barrier()

    def chunk_body(j, carry):
        ch = j * NW + wid

        @pl.when(ch < NCHUNK)
        def _():
            pltpu.sync_copy(dsti.at[ch, 0], idx)
            pltpu.sync_copy(ones_v, cnt_sh.at[idx], add=True)

        return carry

    lax.fori_loop(0, CPW, chunk_body, 0)
    plsc.subcore_barrier()

    rows = pl.ds(s * ROWS_PER_SUB, ROWS_PER_SUB)
    pltpu.sync_copy(cnt_sh.at[rows], cnt_out.at[c, rows])


_sc_count = pl.kernel(
    _sc_count_body,
    out_type=jax.ShapeDtypeStruct((NC, NP, H), jnp.float32),
    mesh=_mesh,
    scratch_types=[
        pltpu.VMEM((CHUNK,), jnp.int32),
        pltpu.VMEM((CHUNK, H), jnp.float32),
        pltpu.VMEM_SHARED((NP, H), jnp.float32),
    ],
)


def _ln_relu(z, g, b):
    mu = jnp.mean(z, axis=-1, keepdims=True)
    zc = z - mu
    var = jnp.mean(zc * zc, axis=-1, keepdims=True)
    return jax.nn.relu(zc * jax.lax.rsqrt(var + 1e-5) * g + b)


def _dot(a, b):
    return jnp.dot(a, b, preferred_element_type=jnp.float32)


def _node_pre_body(feats_ref, nw_ref, nb_ref, wa_ref, wb_ref,
                   h_ref, a_ref, b_ref):
    h = _dot(feats_ref[...], nw_ref[...]) + nb_ref[...]
    h_ref[...] = h
    a_ref[...] = _dot(h, wa_ref[...])
    b_ref[...] = _dot(h, wb_ref[...])


def _node_pre(feats, nw, nb, wa, wb):
    full = lambda shape: pl.BlockSpec(shape, lambda i: (0,) * len(shape))
    return pl.pallas_call(
        _node_pre_body,
        grid=(N // BN,),
        in_specs=[
            pl.BlockSpec((BN, H), lambda i: (i, 0)),
            full((H, H)), full((1, H)), full((H, H2)), full((H, H2)),
        ],
        out_specs=[
            pl.BlockSpec((BN, H), lambda i: (i, 0)),
            pl.BlockSpec((BN, H2), lambda i: (i, 0)),
            pl.BlockSpec((BN, H2), lambda i: (i, 0)),
        ],
        out_shape=[
            jax.ShapeDtypeStruct((N, H), jnp.float32),
            jax.ShapeDtypeStruct((N, H2), jnp.float32),
            jax.ShapeDtypeStruct((N, H2), jnp.float32),
        ],
    )(feats, nw, nb, wa, wb)


def _edge_mlp_body(g_ref, ea_ref, c_ref, cb_ref, w2_ref, b2_ref,
                   g1_ref, be1_ref, g2_ref, be2_ref, out_ref):
    z = g_ref[...] + _dot(ea_ref[...], c_ref[...]) + cb_ref[...]
    m = _ln_relu(z, g1_ref[...], be1_ref[...])
    m2 = _dot(m, w2_ref[...]) + b2_ref[...]
    out_ref[...] = _ln_relu(m2, g2_ref[...], be2_ref[...])


def _edge_mlp(g, ea, cmat, cbias, w2, b2, g1, be1, g2, be2):
    full = lambda shape: pl.BlockSpec(shape, lambda i: (0,) * len(shape))
    return pl.pallas_call(
        _edge_mlp_body,
        grid=(E // BE,),
        in_specs=[
            pl.BlockSpec((BE, H2), lambda i: (i, 0)),
            pl.BlockSpec((BE, 16), lambda i: (i, 0)),
            full((16, H2)), full((1, H2)), full((H2, H)), full((1, H)),
            full((1, H2)), full((1, H2)), full((1, H)), full((1, H)),
        ],
        out_specs=pl.BlockSpec((BE, H), lambda i: (i, 0)),
        out_shape=jax.ShapeDtypeStruct((E, H), jnp.float32),
    )(g, ea, cmat, cbias, w2, b2, g1, be1, g2, be2)


def _make_post_body(with_ab):
    def body(*refs):
        if with_ab:
            (aggp_ref, cntp_ref, h_ref, pw_ref, pb_ref, ng_ref, nb_ref,
             wa_ref, wb_ref, out_ref, a_ref, b_ref) = refs
        else:
            (aggp_ref, cntp_ref, h_ref, pw_ref, pb_ref, ng_ref, nb_ref,
             out_ref) = refs
        agg = aggp_ref[0] + aggp_ref[1]
        cnt = cntp_ref[0, :, 0] + cntp_ref[1, :, 0]
        agg = agg / jnp.maximum(cnt, 1.0)[:, None]
        o = _dot(agg, pw_ref[...]) + pb_ref[...]
        hn = _ln_relu(o, ng_ref[...], nb_ref[...]) + h_ref[...]
        out_ref[...] = hn
        if with_ab:
            a_ref[...] = _dot(hn, wa_ref[...])
            b_ref[...] = _dot(hn, wb_ref[...])
    return body


def _post(aggp, cntp, h, pw, pb, ng, nb, wa=None, wb=None):
    with_ab = wa is not None
    full = lambda shape: pl.BlockSpec(shape, lambda i: (0,) * len(shape))
    in_specs = [
        pl.BlockSpec((NC, BN, H), lambda i: (0, i, 0)),
        pl.BlockSpec((NC, BN, H), lambda i: (0, i, 0)),
        pl.BlockSpec((BN, H), lambda i: (i, 0)),
        full((H, H)), full((1, H)), full((1, H)), full((1, H)),
    ]
    out_specs = [pl.BlockSpec((BN, H), lambda i: (i, 0))]
    out_shape = [jax.ShapeDtypeStruct((N, H), jnp.float32)]
    args = [aggp, cntp, h, pw, pb, ng, nb]
    if with_ab:
        in_specs += [full((H, H2)), full((H, H2))]
        out_specs += [pl.BlockSpec((BN, H2), lambda i: (i, 0)),
                      pl.BlockSpec((BN, H2), lambda i: (i, 0))]
        out_shape += [jax.ShapeDtypeStruct((N, H2), jnp.float32),
                      jax.ShapeDtypeStruct((N, H2), jnp.float32)]
        args += [wa, wb]
    out = pl.pallas_call(
        _make_post_body(with_ab),
        grid=(N // BN,),
        in_specs=in_specs,
        out_specs=out_specs,
        out_shape=out_shape,
    )(*args)
    return out


def kernel(x, pos, edge_attr, params, edge_index, batch):
    feats = jnp.concatenate([x, pos], axis=1)
    src = edge_index[0]
    dst = edge_index[1]
    dsti = dst.reshape(NCHUNK, 1, CHUNK)
    srci = src.reshape(NCHUNK, 1, CHUNK)

    row = lambda v: v.reshape(1, -1)
    wa, wb, cmat, cbias = [], [], [], []
    for lp in params['layers']:
        w1 = lp['W1']
        w1a, w1b, w1c = w1[:H], w1[H:2 * H], w1[2 * H:]
        wa.append(w1a - w1b)
        wb.append(w1b)
        cmat.append(params['edge_W'] @ w1c)
        cbias.append(row(params['edge_b'] @ w1c + lp['b1']))

    h, a, b = _node_pre(feats, params['node_W'], row(params['node_b']),
                        wa[0], wb[0])

    z128 = jnp.zeros((NP, H), jnp.float32)
    cntp = _sc_count(dsti, z128)

    for li, lp in enumerate(params['layers']):
        g = _sc_gather(a, b, dsti, srci)
        m2 = _edge_mlp(g, edge_attr, cmat[li], cbias[li], lp['W2'],
                       row(lp['b2']), row(lp['g1']), row(lp['be1']),
                       row(lp['g2']), row(lp['be2']))
        aggp = _sc_scatter(m2, dsti, z128)
        if li == 0:
            h, a, b = _post(aggp, cntp, h, lp['pW'], row(lp['pb']),
                            row(lp['ng']), row(lp['nb']),
                            wa[1], wb[1])
        else:
            h = _post(aggp, cntp, h, lp['pW'], row(lp['pb']),
                      row(lp['ng']), row(lp['nb']))[0]
    return h
